# Initial kernel scaffold; baseline (speedup 1.0000x reference)
import jax
import jax.numpy as jnp
from jax.experimental import pallas as pl


def kernel(cls_logits_0, reg_preds_0, ctr_logits_0, cls_logits_1, reg_preds_1, ctr_logits_1, cls_logits_2, reg_preds_2, ctr_logits_2, cls_ids, boxes):
    raise NotImplementedError("write your pallas kernel here")


# speedup vs baseline: 4.0309x; 4.0309x over previous
import functools

import jax
import jax.numpy as jnp
from jax import lax
from jax.experimental import pallas as pl
from jax.experimental.pallas import tpu as pltpu
from jax.experimental.pallas import tpu_sc as plsc

_B = 4
_N = 100
_NP = 112
_NG = _NP // 16
_SPB = 8
_HWT = 21504
_INIT = 99999999.0
_RADIO = 1.5
_LEVELS = (
    (128, 128, 8, -1.0, 64.0, 0),
    (64, 64, 16, 64.0, 128.0, 16384),
    (32, 32, 32, 128.0, 999999.0, 20480),
)
_MAXLOC = 2048

_mesh = plsc.VectorSubcoreMesh(
    core_axis_name="c", subcore_axis_name="s", num_cores=2, num_subcores=16
)


def _body(bfl, cfl, cls_out, reg_out, ctr_out,
          bvm, cvm,
          px1, py1, px2, py2, pcl, pcx, pcy, par,
          kx1, ky1, kx2, ky2, kcx, kcy, kar, kidx,
          xsv, ysv, besta, besti, clsv, regv, ctrv, sem):
    wid = lax.axis_index("s") * 2 + lax.axis_index("c")
    b = wid // _SPB
    s = wid % _SPB

    pltpu.sync_copy(bfl, bvm)
    pltpu.sync_copy(cfl, cvm)

    iota = lax.iota(jnp.int32, 16)
    bbase = b * (_N * 4)
    cbase = b * _N
    for g in range(_NG):
        sl = pl.ds(g * 16, 16)
        jl = iota + g * 16
        valid = jl < _N
        ji = jnp.clip(bbase + jl * 4, 0, _B * _N * 4 - 4)
        x1 = jnp.where(valid, plsc.load_gather(bvm, [ji]), 0.0)
        y1 = jnp.where(valid, plsc.load_gather(bvm, [ji + 1]), 0.0)
        x2 = jnp.where(valid, plsc.load_gather(bvm, [ji + 2]), 0.0)
        y2 = jnp.where(valid, plsc.load_gather(bvm, [ji + 3]), 0.0)
        ci = jnp.clip(cbase + jl, 0, _B * _N - 1)
        pcl[sl] = jnp.where(valid, plsc.load_gather(cvm, [ci]), 0)
        px1[sl] = x1
        py1[sl] = y1
        px2[sl] = x2
        py2[sl] = y2
        pcx[sl] = (x1 + x2) * 0.5
        pcy[sl] = (y1 + y2) * 0.5
        par[sl] = (x2 - x1) * (y2 - y1)

    for (h, w, stride, rmin, rmax, lvl_off) in _LEVELS:
        locs = (h * w) // _SPB
        nv = locs // 16
        rows = locs // w
        radius = _RADIO * stride
        inv = 1.0 / stride
        half = float(stride // 2)
        row_lo = s * rows
        ylo = row_lo.astype(jnp.float32) * float(stride) + half
        yhi = ylo + float((rows - 1) * stride)

        @plsc.parallel_loop(0, nv, unroll=4)
        def _fill(v):
            sl = pl.ds(v * 16, 16)
            g0 = s * locs + v * 16
            col = (g0 % w) + iota
            row = g0 // w
            xsv[sl] = col.astype(jnp.float32) * float(stride) + half
            ysv[sl] = jnp.full((16,), half, jnp.float32) + (
                row.astype(jnp.float32) * float(stride)
            )
            besta[sl] = jnp.full((16,), _INIT, jnp.float32)
            besti[sl] = jnp.zeros((16,), jnp.int32)

        cnt = jnp.int32(0)
        for g in range(_NG):
            sl = pl.ds(g * 16, 16)
            x1 = px1[sl]
            y1 = py1[sl]
            x2 = px2[sl]
            y2 = py2[sl]
            cxv = pcx[sl]
            cyv = pcy[sl]
            maxd = jnp.maximum(x2 - x1, y2 - y1)
            keep = (cyv >= ylo - radius) & (cyv <= yhi + radius)
            keep &= (y1 < yhi) & (y2 > ylo)
            if rmin > 0.0:
                keep &= maxd > rmin
            if rmax < 99999.0:
                keep &= maxd * 0.5 <= rmax
            kpi = keep.astype(jnp.int32)
            posn = jnp.clip(cnt + plsc.cumsum(kpi) - 1, 0, _NP - 1)
            plsc.store_scatter(kx1, [posn], x1, mask=keep)
            plsc.store_scatter(ky1, [posn], y1, mask=keep)
            plsc.store_scatter(kx2, [posn], x2, mask=keep)
            plsc.store_scatter(ky2, [posn], y2, mask=keep)
            plsc.store_scatter(kcx, [posn], cxv, mask=keep)
            plsc.store_scatter(kcy, [posn], cyv, mask=keep)
            plsc.store_scatter(kar, [posn], par[sl], mask=keep)
            plsc.store_scatter(kidx, [posn], iota + g * 16, mask=keep)
            cnt = cnt + jnp.sum(kpi)

        def box_step(j, _):
            jf = jnp.full((16,), j, jnp.int32)
            x1v = plsc.load_gather(kx1, [jf])
            y1v = plsc.load_gather(ky1, [jf])
            x2v = plsc.load_gather(kx2, [jf])
            y2v = plsc.load_gather(ky2, [jf])
            cxv = plsc.load_gather(kcx, [jf])
            cyv = plsc.load_gather(kcy, [jf])
            arv = plsc.load_gather(kar, [jf])
            iv = plsc.load_gather(kidx, [jf])

            vy = (cyv - (radius + half + 1.0)) * inv
            ry0 = vy.astype(jnp.int32)
            ry0 = ry0 - (ry0.astype(jnp.float32) > vy).astype(jnp.int32)
            vx = (cxv - (radius + half + 1.0)) * inv
            cols = vx.astype(jnp.int32) - 1 + iota
            colc = jnp.clip(cols, 0, w - 1)
            x = cols.astype(jnp.float32) * float(stride) + half
            colok = (cols >= 0) & (cols < w)
            dxa = jnp.abs(x - cxv)
            l = x - x1v
            r = x2v - x
            lrok = colok & (jnp.minimum(l, r) > 0.0)
            lrmx = jnp.maximum(l, r)

            pend = []
            for dr in range(5):
                rowv = ry0 + dr
                yv = rowv.astype(jnp.float32) * float(stride) + half
                t = yv - y1v
                bb = y2v - yv
                mn = jnp.minimum(t, bb)
                mx = jnp.maximum(lrmx, jnp.maximum(t, bb))
                ok = lrok & (mn > 0.0)
                if rmin > 0.0:
                    ok &= mx > rmin
                if rmax < 99999.0:
                    ok &= mx <= rmax
                ok &= jnp.maximum(dxa, jnp.abs(yv - cyv)) <= radius
                ok &= (rowv >= row_lo) & (rowv < row_lo + rows)
                lidx = jnp.clip((rowv - row_lo) * w + colc, 0, locs - 1)
                ba = plsc.load_gather(besta, [lidx])
                pend.append((ok & (arv < ba), lidx))
            for upd, lidx in pend:
                plsc.store_scatter(besta, [lidx], arv, mask=upd)
                plsc.store_scatter(besti, [lidx], iv, mask=upd)
            return None

        lax.fori_loop(0, cnt, box_step, None)

        @plsc.parallel_loop(0, nv, unroll=2)
        def _finish(v):
            sl = pl.ds(v * 16, 16)
            ba = besta[sl]
            bi = besti[sl]
            pos = ba < _INIT
            x1v = plsc.load_gather(px1, [bi])
            y1v = plsc.load_gather(py1, [bi])
            x2v = plsc.load_gather(px2, [bi])
            y2v = plsc.load_gather(py2, [bi])
            clv = plsc.load_gather(pcl, [bi])
            x = xsv[sl]
            y = ysv[sl]
            l = (x - x1v) * inv
            t = (y - y1v) * inv
            r = (x2v - x) * inv
            bb = (y2v - y) * inv
            lrmin = jnp.minimum(l, r)
            lrmax = jnp.maximum(l, r)
            tbmin = jnp.minimum(t, bb)
            tbmax = jnp.maximum(t, bb)
            num = lrmin * tbmin
            den = jnp.maximum(lrmax * tbmax, 1e-10)
            ratio = jnp.where(pos, num / den, 1.0)
            rc = jnp.maximum(ratio, 1e-12)
            bits = lax.bitcast_convert_type(rc, jnp.int32)
            yk = lax.bitcast_convert_type(
                jnp.int32(0x5F3759DF) - (bits >> 1), jnp.float32
            )
            for _i in range(3):
                yk = yk * (1.5 - 0.5 * rc * yk * yk)
            sq = rc * yk
            ctrv[sl] = jnp.where(pos, sq, -1.0)
            clsv[sl] = jnp.where(pos, clv, 0)
            idx4 = iota * 4 + (v * 64)
            store = plsc.store_scatter
            store(regv, [idx4], jnp.where(pos, l, -1.0))
            store(regv, [idx4 + 1], jnp.where(pos, t, -1.0))
            store(regv, [idx4 + 2], jnp.where(pos, r, -1.0))
            store(regv, [idx4 + 3], jnp.where(pos, bb, -1.0))

        off = lvl_off + s * locs
        pltpu.sync_copy(clsv.at[pl.ds(0, locs)], cls_out.at[b, pl.ds(off, locs)])
        pltpu.sync_copy(ctrv.at[pl.ds(0, locs)], ctr_out.at[b, pl.ds(off, locs)])
        pltpu.sync_copy(
            regv.at[pl.ds(0, locs * 4)], reg_out.at[b, pl.ds(off * 4, locs * 4)]
        )


def _make_sc_call(interpret=False):
    return pl.kernel(
        _body,
        out_type=(
            jax.ShapeDtypeStruct((_B, _HWT), jnp.int32),
            jax.ShapeDtypeStruct((_B, _HWT * 4), jnp.float32),
            jax.ShapeDtypeStruct((_B, _HWT), jnp.float32),
        ),
        mesh=_mesh,
        scratch_types=[
            pltpu.VMEM((_B * _N * 4,), jnp.float32),
            pltpu.VMEM((_B * _N,), jnp.int32),
            pltpu.VMEM((_NP,), jnp.float32),
            pltpu.VMEM((_NP,), jnp.float32),
            pltpu.VMEM((_NP,), jnp.float32),
            pltpu.VMEM((_NP,), jnp.float32),
            pltpu.VMEM((_NP,), jnp.int32),
            pltpu.VMEM((_NP,), jnp.float32),
            pltpu.VMEM((_NP,), jnp.float32),
            pltpu.VMEM((_NP,), jnp.float32),
            pltpu.VMEM((_NP,), jnp.float32),
            pltpu.VMEM((_NP,), jnp.float32),
            pltpu.VMEM((_NP,), jnp.float32),
            pltpu.VMEM((_NP,), jnp.float32),
            pltpu.VMEM((_NP,), jnp.float32),
            pltpu.VMEM((_NP,), jnp.float32),
            pltpu.VMEM((_NP,), jnp.float32),
            pltpu.VMEM((_NP,), jnp.int32),
            pltpu.VMEM((_MAXLOC,), jnp.float32),
            pltpu.VMEM((_MAXLOC,), jnp.float32),
            pltpu.VMEM((_MAXLOC,), jnp.float32),
            pltpu.VMEM((_MAXLOC,), jnp.int32),
            pltpu.VMEM((_MAXLOC,), jnp.int32),
            pltpu.VMEM((_MAXLOC * 4,), jnp.float32),
            pltpu.VMEM((_MAXLOC,), jnp.float32),
            pltpu.SemaphoreType.DMA,
        ],
        compiler_params=pltpu.CompilerParams(
            needs_layout_passes=False,
            skip_device_barrier=True,
            disable_bounds_checks=True,
            disable_semaphore_checks=True,
        ),
        interpret=interpret,
    )


_sc_call = _make_sc_call()


@jax.jit
def _run(cls_ids, boxes):
    bfl = boxes.astype(jnp.float32).reshape(-1)
    cfl = cls_ids.astype(jnp.int32).reshape(-1)
    cls_f, reg_f, ctr_f = _sc_call(bfl, cfl)
    return (
        cls_f[..., None],
        reg_f.reshape(_B, _HWT, 4),
        ctr_f[..., None],
    )


def kernel(cls_logits_0, reg_preds_0, ctr_logits_0, cls_logits_1, reg_preds_1,
           ctr_logits_1, cls_logits_2, reg_preds_2, ctr_logits_2, cls_ids, boxes):
    return _run(cls_ids, boxes)
